# Initial kernel scaffold; baseline (speedup 1.0000x reference)
import jax
import jax.numpy as jnp
from jax.experimental import pallas as pl


def kernel(X):
    raise NotImplementedError("write your pallas kernel here")


# speedup vs baseline: 2.0023x; 2.0023x over previous
import functools

import jax
import jax.numpy as jnp
from jax import lax
from jax.experimental import pallas as pl
from jax.experimental.pallas import tpu as pltpu
from jax.experimental.pallas import tpu_sc as plsc

N_ROWS = 65536
N_COLS = 60
N_DATA = 30
N_SEG = 31
N_NEW = 33
LANES = 16

NUM_CORES = 2
NUM_SUBCORES = 16
NW = NUM_CORES * NUM_SUBCORES
ROWS_PER_W = N_ROWS // NW
CHUNK = 128
GROUPS = CHUNK // LANES
NCHUNKS = ROWS_PER_W // CHUNK
NBUF = 2


def _interp_body(x_hbm, out_hbm, x_v0, x_v1, out_v0, out_v1, hist,
                 m_buf, b_buf, sem_in, sem_out):
    x_v = [x_v0, x_v1]
    out_v = [out_v0, out_v1]
    wid = lax.axis_index("s") * NUM_CORES + lax.axis_index("c")
    lane = lax.iota(jnp.int32, LANES)
    row0_w = wid * ROWS_PER_W

    def process_group(g, _, xc, oc):
        rbase60 = (g * LANES + lane) * N_COLS
        rbase33 = (g * LANES + lane) * N_NEW

        for k in range(N_NEW):
            hist[pl.ds(k * LANES, LANES)] = jnp.zeros((LANES,), jnp.int32)

        ones = jnp.ones((LANES,), jnp.int32)

        def hist_knot(j, _):
            xj = plsc.load_gather(xc, [rbase60 + j])
            t = xj * 32.0
            ti = t.astype(jnp.int32)
            c = ti + (t > ti.astype(jnp.float32)).astype(jnp.int32)
            plsc.addupdate_scatter(hist, [c * LANES + lane], ones)
            return 0

        lax.fori_loop(0, N_DATA, hist_knot, 0)

        def seg(j, carry):
            px, py = carry
            cx = plsc.load_gather(xc, [rbase60 + j])
            cy = plsc.load_gather(xc, [rbase60 + (N_DATA + j)])
            m = (cy - py) / (cx - px)
            b = py - m * px
            m_buf[pl.ds(j * LANES, LANES)] = m
            b_buf[pl.ds(j * LANES, LANES)] = b
            return cx, cy

        zero_v = jnp.zeros((LANES,), jnp.float32)
        px, py = lax.fori_loop(0, N_DATA, seg, (zero_v, zero_v))
        m = (1.0 - py) / (1.0 - px)
        b = py - m * px
        m_buf[pl.ds(N_DATA * LANES, LANES)] = m
        b_buf[pl.ds(N_DATA * LANES, LANES)] = b

        def emit(k, run):
            run = run + hist[pl.ds(k * LANES, LANES)]
            idx = jnp.minimum(run - 1, N_SEG - 1)
            gi = idx * LANES + lane
            ms = plsc.load_gather(m_buf, [gi])
            bs = plsc.load_gather(b_buf, [gi])
            nx = k.astype(jnp.float32) * 0.03125
            res = ms * nx + bs
            plsc.store_scatter(oc, [rbase33 + k], res)
            return run

        lax.fori_loop(0, N_NEW, emit, jnp.ones((LANES,), jnp.int32))
        return 0

    for b in range(NBUF):
        pltpu.async_copy(
            x_hbm.at[pl.ds((row0_w + b * CHUNK) * N_COLS, CHUNK * N_COLS)],
            x_v[b], sem_in.at[b])

    def chunk_round(c2, _):
        for b in range(NBUF):
            c = c2 * NBUF + b
            row0 = row0_w + c * CHUNK
            pltpu.make_async_copy(
                x_hbm.at[pl.ds(row0 * N_COLS, CHUNK * N_COLS)], x_v[b],
                sem_in.at[b]).wait()

            @pl.when(c >= NBUF)
            def _():
                pltpu.make_async_copy(
                    out_v[b],
                    out_hbm.at[pl.ds((row0 - NBUF * CHUNK) * N_NEW,
                                     CHUNK * N_NEW)],
                    sem_out.at[b]).wait()

            lax.fori_loop(
                0, GROUPS,
                functools.partial(process_group, xc=x_v[b],
                                  oc=out_v[b]),
                0)

            @pl.when(c + NBUF < NCHUNKS)
            def _():
                pltpu.async_copy(
                    x_hbm.at[pl.ds((row0 + NBUF * CHUNK) * N_COLS,
                                   CHUNK * N_COLS)],
                    x_v[b], sem_in.at[b])

            pltpu.async_copy(
                out_v[b], out_hbm.at[pl.ds(row0 * N_NEW, CHUNK * N_NEW)],
                sem_out.at[b])
        return 0

    lax.fori_loop(0, NCHUNKS // NBUF, chunk_round, 0)

    for b in range(NBUF):
        row0 = row0_w + (NCHUNKS - NBUF + b) * CHUNK
        pltpu.make_async_copy(
            out_v[b], out_hbm.at[pl.ds(row0 * N_NEW, CHUNK * N_NEW)],
            sem_out.at[b]).wait()


def _build():
    mesh = plsc.VectorSubcoreMesh(core_axis_name="c", subcore_axis_name="s")
    return pl.kernel(
        _interp_body,
        mesh=mesh,
        compiler_params=pltpu.CompilerParams(needs_layout_passes=False),
        out_type=jax.ShapeDtypeStruct((N_ROWS * N_NEW,), jnp.float32),
        scratch_types=[
            pltpu.VMEM((CHUNK * N_COLS,), jnp.float32),
            pltpu.VMEM((CHUNK * N_COLS,), jnp.float32),
            pltpu.VMEM((CHUNK * N_NEW,), jnp.float32),
            pltpu.VMEM((CHUNK * N_NEW,), jnp.float32),
            pltpu.VMEM((N_NEW * LANES,), jnp.int32),
            pltpu.VMEM((N_SEG * LANES,), jnp.float32),
            pltpu.VMEM((N_SEG * LANES,), jnp.float32),
            pltpu.SemaphoreType.DMA((NBUF,)),
            pltpu.SemaphoreType.DMA((NBUF,)),
        ],
    )


@jax.jit
def kernel(X):
    flat = _build()(X.reshape(N_ROWS * N_COLS))
    return flat.reshape(N_ROWS, N_NEW)


# speedup vs baseline: 2.4225x; 1.2099x over previous
import functools

import jax
import jax.numpy as jnp
from jax import lax
from jax.experimental import pallas as pl
from jax.experimental.pallas import tpu as pltpu
from jax.experimental.pallas import tpu_sc as plsc

N_ROWS = 65536
N_COLS = 60
N_DATA = 30
N_SEG = 31
N_NEW = 33
LANES = 16

NUM_CORES = 2
NUM_SUBCORES = 16
NW = NUM_CORES * NUM_SUBCORES
ROWS_PER_W = N_ROWS // NW
CHUNK = 128
GROUPS = CHUNK // LANES
NCHUNKS = ROWS_PER_W // CHUNK
NBUF = 2


def _interp_body(x_hbm, out_hbm, x_v0, x_v1, out_v0, out_v1, hist,
                 m_buf, b_buf, sem_in, sem_out):
    x_v = [x_v0, x_v1]
    out_v = [out_v0, out_v1]
    wid = lax.axis_index("s") * NUM_CORES + lax.axis_index("c")
    lane = lax.iota(jnp.int32, LANES)
    row0_w = wid * ROWS_PER_W

    def process_group(g, _, xc, oc):
        rbase60 = (g * LANES + lane) * N_COLS
        rbase33 = (g * LANES + lane) * N_NEW

        zero_i = jnp.zeros((LANES,), jnp.int32)
        for k in range(N_NEW):
            hist[pl.ds(k * LANES, LANES)] = zero_i

        ones = jnp.ones((LANES,), jnp.int32)
        px = jnp.zeros((LANES,), jnp.float32)
        py = jnp.zeros((LANES,), jnp.float32)
        for j in range(N_DATA):
            cx = plsc.load_gather(xc, [rbase60 + j])
            cy = plsc.load_gather(xc, [rbase60 + (N_DATA + j)])
            t = cx * 32.0
            ti = t.astype(jnp.int32)
            c = ti + (t > ti.astype(jnp.float32)).astype(jnp.int32)
            plsc.addupdate_scatter(hist, [c * LANES + lane], ones)
            m = (cy - py) / (cx - px)
            b = py - m * px
            m_buf[pl.ds(j * LANES, LANES)] = m
            b_buf[pl.ds(j * LANES, LANES)] = b
            px, py = cx, cy
        m = (1.0 - py) / (1.0 - px)
        b = py - m * px
        m_buf[pl.ds(N_DATA * LANES, LANES)] = m
        b_buf[pl.ds(N_DATA * LANES, LANES)] = b

        run = ones
        for k in range(N_NEW):
            run = run + hist[pl.ds(k * LANES, LANES)]
            idx = jnp.minimum(run - 1, N_SEG - 1)
            gi = idx * LANES + lane
            ms = plsc.load_gather(m_buf, [gi])
            bs = plsc.load_gather(b_buf, [gi])
            res = ms * (k * 0.03125) + bs
            plsc.store_scatter(oc, [rbase33 + k], res)
        return 0

    for b in range(NBUF):
        pltpu.async_copy(
            x_hbm.at[pl.ds((row0_w + b * CHUNK) * N_COLS, CHUNK * N_COLS)],
            x_v[b], sem_in.at[b])

    def chunk_round(c2, _):
        for b in range(NBUF):
            c = c2 * NBUF + b
            row0 = row0_w + c * CHUNK
            pltpu.make_async_copy(
                x_hbm.at[pl.ds(row0 * N_COLS, CHUNK * N_COLS)], x_v[b],
                sem_in.at[b]).wait()

            @pl.when(c >= NBUF)
            def _():
                pltpu.make_async_copy(
                    out_v[b],
                    out_hbm.at[pl.ds((row0 - NBUF * CHUNK) * N_NEW,
                                     CHUNK * N_NEW)],
                    sem_out.at[b]).wait()

            lax.fori_loop(
                0, GROUPS,
                functools.partial(process_group, xc=x_v[b],
                                  oc=out_v[b]),
                0)

            @pl.when(c + NBUF < NCHUNKS)
            def _():
                pltpu.async_copy(
                    x_hbm.at[pl.ds((row0 + NBUF * CHUNK) * N_COLS,
                                   CHUNK * N_COLS)],
                    x_v[b], sem_in.at[b])

            pltpu.async_copy(
                out_v[b], out_hbm.at[pl.ds(row0 * N_NEW, CHUNK * N_NEW)],
                sem_out.at[b])
        return 0

    lax.fori_loop(0, NCHUNKS // NBUF, chunk_round, 0)

    for b in range(NBUF):
        row0 = row0_w + (NCHUNKS - NBUF + b) * CHUNK
        pltpu.make_async_copy(
            out_v[b], out_hbm.at[pl.ds(row0 * N_NEW, CHUNK * N_NEW)],
            sem_out.at[b]).wait()


def _build():
    mesh = plsc.VectorSubcoreMesh(core_axis_name="c", subcore_axis_name="s")
    return pl.kernel(
        _interp_body,
        mesh=mesh,
        compiler_params=pltpu.CompilerParams(needs_layout_passes=False),
        out_type=jax.ShapeDtypeStruct((N_ROWS * N_NEW,), jnp.float32),
        scratch_types=[
            pltpu.VMEM((CHUNK * N_COLS,), jnp.float32),
            pltpu.VMEM((CHUNK * N_COLS,), jnp.float32),
            pltpu.VMEM((CHUNK * N_NEW,), jnp.float32),
            pltpu.VMEM((CHUNK * N_NEW,), jnp.float32),
            pltpu.VMEM((N_NEW * LANES,), jnp.int32),
            pltpu.VMEM((N_SEG * LANES,), jnp.float32),
            pltpu.VMEM((N_SEG * LANES,), jnp.float32),
            pltpu.SemaphoreType.DMA((NBUF,)),
            pltpu.SemaphoreType.DMA((NBUF,)),
        ],
    )


@jax.jit
def kernel(X):
    flat = _build()(X.reshape(N_ROWS * N_COLS))
    return flat.reshape(N_ROWS, N_NEW)
